# initial kernel scaffold (unmeasured)
import jax
import jax.numpy as jnp
from jax import lax
from jax.experimental import pallas as pl
from jax.experimental.pallas import tpu as pltpu

N_DEV = 16
MB = 256
NH = 4096
F32 = jnp.float32
BF16 = jnp.bfloat16


def kernel(x, w_mat):
    m, k_sh = x.shape
    _, n = w_mat.shape

    def body(x_ref, w_ref, out_ref,
             x_bf, w_bf,
             acc_cw, acc_ccw,
             send_cw, recv_cw, send_ccw, recv_ccw,
             ssem_cw, rsem_cw, ssem_ccw, rsem_ccw,
             credit_cw, credit_ccw,
             amax_src, amax_buf, a_ssem, a_rsem):
        d = lax.axis_index("i")
        right = jnp.mod(d + 1, N_DEV)
        left = jnp.mod(d - 1, N_DEV)

        barrier = pltpu.get_barrier_semaphore()
        pl.semaphore_signal(barrier, inc=1, device_id=(left,),
                            device_id_type=pl.DeviceIdType.MESH)
        pl.semaphore_signal(barrier, inc=1, device_id=(right,),
                            device_id_type=pl.DeviceIdType.MESH)
        pl.semaphore_wait(barrier, 2)

        x_bf[...] = x_ref[...].astype(BF16)
        w_bf[...] = w_ref[...].astype(BF16)

        def partial(b, half):
            xb = x_bf[pl.ds(b * MB, MB), :]
            wb = w_bf[:, 0:NH] if half == 0 else w_bf[:, NH:2 * NH]
            return jnp.dot(xb, wb, preferred_element_type=F32)

        acc_cw[...] = partial(jnp.mod(d - 1, N_DEV), 0)
        acc_ccw[...] = partial(jnp.mod(d + 1, N_DEV), 1)

        desc_cw = {}
        desc_ccw = {}
        for s in range(N_DEV - 1):
            slot = s % 2
            if s >= 2:
                desc_cw[slot].wait_send()
                desc_ccw[slot].wait_send()
            send_cw[slot] = acc_cw[...].astype(BF16)
            send_ccw[slot] = acc_ccw[...].astype(BF16)
            if s >= 2:
                pl.semaphore_wait(credit_cw, 1)
                pl.semaphore_wait(credit_ccw, 1)
            rc = pltpu.make_async_remote_copy(
                src_ref=send_cw.at[slot],
                dst_ref=recv_cw.at[slot],
                send_sem=ssem_cw.at[slot],
                recv_sem=rsem_cw.at[slot],
                device_id=(right,),
                device_id_type=pl.DeviceIdType.MESH,
            )
            rc.start()
            rcc = pltpu.make_async_remote_copy(
                src_ref=send_ccw.at[slot],
                dst_ref=recv_ccw.at[slot],
                send_sem=ssem_ccw.at[slot],
                recv_sem=rsem_ccw.at[slot],
                device_id=(left,),
                device_id_type=pl.DeviceIdType.MESH,
            )
            rcc.start()
            p_cw = partial(jnp.mod(d - 2 - s, N_DEV), 0)
            p_ccw = partial(jnp.mod(d + 2 + s, N_DEV), 1)
            rc.wait_recv()
            acc_cw[...] = recv_cw[slot].astype(F32) + p_cw
            rcc.wait_recv()
            acc_ccw[...] = recv_ccw[slot].astype(F32) + p_ccw
            if s + 2 <= N_DEV - 2:
                pl.semaphore_signal(credit_cw, inc=1, device_id=(left,),
                                    device_id_type=pl.DeviceIdType.MESH)
                pl.semaphore_signal(credit_ccw, inc=1, device_id=(right,),
                                    device_id_type=pl.DeviceIdType.MESH)
            desc_cw[slot] = rc
            desc_ccw[slot] = rcc
        for slot in (0, 1):
            desc_cw[slot].wait_send()
            desc_ccw[slot].wait_send()

        y_cw = jnp.maximum(acc_cw[...], 0.0)
        y_ccw = jnp.maximum(acc_ccw[...], 0.0)
        lmax = jnp.maximum(jnp.max(y_cw), jnp.max(y_ccw))
        amax_src[...] = jnp.full((1, 8, 128), lmax, F32)
        amax_buf[pl.ds(d, 1)] = amax_src[...]

        amax_sends = []
        for kk in range(1, N_DEV):
            j = jnp.mod(d + kk, N_DEV)
            rd = pltpu.make_async_remote_copy(
                src_ref=amax_src,
                dst_ref=amax_buf.at[pl.ds(d, 1)],
                send_sem=a_ssem.at[kk - 1],
                recv_sem=a_rsem.at[d],
                device_id=(j,),
                device_id_type=pl.DeviceIdType.MESH,
            )
            rd.start()
            amax_sends.append(rd)
        for kk in range(1, N_DEV):
            i = jnp.mod(d + kk, N_DEV)
            rr = pltpu.make_async_remote_copy(
                src_ref=amax_src,
                dst_ref=amax_buf.at[pl.ds(i, 1)],
                send_sem=a_ssem.at[0],
                recv_sem=a_rsem.at[i],
                device_id=(d,),
                device_id_type=pl.DeviceIdType.MESH,
            )
            rr.wait_recv()
        for rd in amax_sends:
            rd.wait_send()

        gmax = jnp.max(amax_buf[...])
        scale = gmax / 448.0
        q_cw = (y_cw / scale).astype(jnp.float8_e4m3fn)
        q_ccw = (y_ccw / scale).astype(jnp.float8_e4m3fn)
        out_ref[:, 0:NH] = q_cw.astype(F32) * scale
        out_ref[:, NH:2 * NH] = q_ccw.astype(F32) * scale

    return pl.pallas_call(
        body,
        out_shape=jax.ShapeDtypeStruct((MB, n), F32),
        in_specs=[
            pl.BlockSpec(memory_space=pltpu.VMEM),
            pl.BlockSpec(memory_space=pltpu.VMEM),
        ],
        out_specs=pl.BlockSpec(memory_space=pltpu.VMEM),
        scratch_shapes=[
            pltpu.VMEM((4096, 256), BF16),
            pltpu.VMEM((256, 8192), BF16),
            pltpu.VMEM((MB, NH), F32),
            pltpu.VMEM((MB, NH), F32),
            pltpu.VMEM((2, MB, NH), BF16),
            pltpu.VMEM((2, MB, NH), BF16),
            pltpu.VMEM((2, MB, NH), BF16),
            pltpu.VMEM((2, MB, NH), BF16),
            pltpu.SemaphoreType.DMA((2,)),
            pltpu.SemaphoreType.DMA((2,)),
            pltpu.SemaphoreType.DMA((2,)),
            pltpu.SemaphoreType.DMA((2,)),
            pltpu.SemaphoreType.REGULAR,
            pltpu.SemaphoreType.REGULAR,
            pltpu.VMEM((1, 8, 128), F32),
            pltpu.VMEM((N_DEV, 8, 128), F32),
            pltpu.SemaphoreType.DMA((N_DEV - 1,)),
            pltpu.SemaphoreType.DMA((N_DEV,)),
        ],
        compiler_params=pltpu.CompilerParams(collective_id=0),
    )(x, w_mat)


# baseline (device time: 421725 ns/iter reference)
import jax
import jax.numpy as jnp
from jax import lax
from jax.experimental import pallas as pl
from jax.experimental.pallas import tpu as pltpu

N_DEV = 16
MB = 256
NH = 4096
SCALE_FP = 2048.0
F32 = jnp.float32
BF16 = jnp.bfloat16
I16 = jnp.int16


def kernel(x, w_mat):
    m, k_sh = x.shape
    _, n = w_mat.shape

    def body(x_ref, w_ref, out_ref,
             x_bf,
             acc_cw, acc_ccw, recv_cw, recv_ccw,
             ssem_cw, rsem_cw, ssem_ccw, rsem_ccw,
             credit_cw, credit_ccw,
             amax_src, amax_buf, a_ssem, a_rsem):
        d = lax.axis_index("i")
        right = jnp.mod(d + 1, N_DEV)
        left = jnp.mod(d - 1, N_DEV)

        barrier = pltpu.get_barrier_semaphore()
        pl.semaphore_signal(barrier, inc=1, device_id=(left,),
                            device_id_type=pl.DeviceIdType.MESH)
        pl.semaphore_signal(barrier, inc=1, device_id=(right,),
                            device_id_type=pl.DeviceIdType.MESH)
        pl.semaphore_wait(barrier, 2)

        x_bf[...] = x_ref[...].astype(BF16)

        def partial(b, half):
            xb = x_bf[pl.ds(b * MB, MB), :]
            wb = w_ref[:, 0:NH] if half == 0 else w_ref[:, NH:2 * NH]
            return jnp.dot(xb, wb.astype(BF16), preferred_element_type=F32)

        def quant16(p):
            return jnp.round(p * SCALE_FP).astype(I16)

        acc_cw[0] = quant16(partial(jnp.mod(d - 1, N_DEV), 0))
        acc_ccw[0] = quant16(partial(jnp.mod(d + 1, N_DEV), 1))

        desc_cw = {}
        desc_ccw = {}
        for s in range(N_DEV - 1):
            a = s % 2
            if s >= 2:
                pl.semaphore_wait(credit_cw, 1)
                pl.semaphore_wait(credit_ccw, 1)
            rc = pltpu.make_async_remote_copy(
                src_ref=acc_cw.at[a],
                dst_ref=recv_cw.at[a],
                send_sem=ssem_cw.at[a],
                recv_sem=rsem_cw.at[a],
                device_id=(right,),
                device_id_type=pl.DeviceIdType.MESH,
            )
            rc.start()
            rcc = pltpu.make_async_remote_copy(
                src_ref=acc_ccw.at[a],
                dst_ref=recv_ccw.at[a],
                send_sem=ssem_ccw.at[a],
                recv_sem=rsem_ccw.at[a],
                device_id=(left,),
                device_id_type=pl.DeviceIdType.MESH,
            )
            rcc.start()
            p_cw = partial(jnp.mod(d - 2 - s, N_DEV), 0)
            p_ccw = partial(jnp.mod(d + 2 + s, N_DEV), 1)
            if s >= 1:
                desc_cw[1 - a].wait_send()
                desc_ccw[1 - a].wait_send()
            last = s == N_DEV - 2
            rc.wait_recv()
            if last:
                out_ref[:, 0:NH] = jnp.maximum(
                    recv_cw[a].astype(F32) * (1.0 / SCALE_FP) + p_cw, 0.0)
            else:
                acc_cw[1 - a] = recv_cw[a] + quant16(p_cw)
            rcc.wait_recv()
            if last:
                out_ref[:, NH:2 * NH] = jnp.maximum(
                    recv_ccw[a].astype(F32) * (1.0 / SCALE_FP) + p_ccw, 0.0)
            else:
                acc_ccw[1 - a] = recv_ccw[a] + quant16(p_ccw)
            if s + 2 <= N_DEV - 2:
                pl.semaphore_signal(credit_cw, inc=1, device_id=(left,),
                                    device_id_type=pl.DeviceIdType.MESH)
                pl.semaphore_signal(credit_ccw, inc=1, device_id=(right,),
                                    device_id_type=pl.DeviceIdType.MESH)
            desc_cw[a] = rc
            desc_ccw[a] = rcc
        final_a = (N_DEV - 2) % 2
        desc_cw[final_a].wait_send()
        desc_ccw[final_a].wait_send()

        CH = 2048
        lmax = jnp.float32(0.0)
        for c in range(0, 2 * NH, CH):
            lmax = jnp.maximum(lmax, jnp.max(out_ref[:, c:c + CH]))
        amax_src[...] = jnp.full((1, 8, 128), lmax, F32)
        amax_buf[pl.ds(d, 1)] = amax_src[...]

        amax_sends = []
        for kk in range(1, N_DEV):
            j = jnp.mod(d + kk, N_DEV)
            rd = pltpu.make_async_remote_copy(
                src_ref=amax_src,
                dst_ref=amax_buf.at[pl.ds(d, 1)],
                send_sem=a_ssem.at[kk - 1],
                recv_sem=a_rsem.at[d],
                device_id=(j,),
                device_id_type=pl.DeviceIdType.MESH,
            )
            rd.start()
            amax_sends.append(rd)
        for kk in range(1, N_DEV):
            i = jnp.mod(d + kk, N_DEV)
            rr = pltpu.make_async_remote_copy(
                src_ref=amax_src,
                dst_ref=amax_buf.at[pl.ds(i, 1)],
                send_sem=a_ssem.at[0],
                recv_sem=a_rsem.at[i],
                device_id=(d,),
                device_id_type=pl.DeviceIdType.MESH,
            )
            rr.wait_recv()
        for rd in amax_sends:
            rd.wait_send()

        gmax = jnp.max(amax_buf[...])
        scale = gmax / 448.0
        for c in range(0, 2 * NH, CH):
            q = (out_ref[:, c:c + CH] / scale).astype(jnp.float8_e4m3fn)
            out_ref[:, c:c + CH] = q.astype(F32) * scale

    return pl.pallas_call(
        body,
        out_shape=jax.ShapeDtypeStruct((MB, n), F32),
        in_specs=[
            pl.BlockSpec(memory_space=pltpu.VMEM),
            pl.BlockSpec(memory_space=pltpu.VMEM),
        ],
        out_specs=pl.BlockSpec(memory_space=pltpu.VMEM),
        scratch_shapes=[
            pltpu.VMEM((4096, 256), BF16),
            pltpu.VMEM((2, MB, NH), I16),
            pltpu.VMEM((2, MB, NH), I16),
            pltpu.VMEM((2, MB, NH), I16),
            pltpu.VMEM((2, MB, NH), I16),
            pltpu.SemaphoreType.DMA((2,)),
            pltpu.SemaphoreType.DMA((2,)),
            pltpu.SemaphoreType.DMA((2,)),
            pltpu.SemaphoreType.DMA((2,)),
            pltpu.SemaphoreType.REGULAR,
            pltpu.SemaphoreType.REGULAR,
            pltpu.VMEM((1, 8, 128), F32),
            pltpu.VMEM((N_DEV, 8, 128), F32),
            pltpu.SemaphoreType.DMA((N_DEV - 1,)),
            pltpu.SemaphoreType.DMA((N_DEV,)),
        ],
        compiler_params=pltpu.CompilerParams(collective_id=0),
    )(x, w_mat)


# device time: 413819 ns/iter; 1.0191x vs baseline; 1.0191x over previous
import jax
import jax.numpy as jnp
from jax import lax
from jax.experimental import pallas as pl
from jax.experimental.pallas import tpu as pltpu

N_DEV = 16
MB = 256
SB = 128
NH = 4096
SCALE_FP = 2048.0
F32 = jnp.float32
BF16 = jnp.bfloat16
I16 = jnp.int16


def kernel(x, w_mat):
    m, k_sh = x.shape
    _, n = w_mat.shape

    def body(x_ref, w_ref, out_ref, x_bf, w_bf,
             acc_cwA, acc_cwB, acc_ccwA, acc_ccwB,
             recv_cwA, recv_cwB, recv_ccwA, recv_ccwB,
             ssem_cwA, rsem_cwA, ssem_cwB, rsem_cwB,
             ssem_ccwA, rsem_ccwA, ssem_ccwB, rsem_ccwB,
             cred_cwA, cred_cwB, cred_ccwA, cred_ccwB,
             amax_src, amax_buf, a_ssem, a_rsem):
        d = lax.axis_index("i")
        right = jnp.mod(d + 1, N_DEV)
        left = jnp.mod(d - 1, N_DEV)

        barrier = pltpu.get_barrier_semaphore()
        pl.semaphore_signal(barrier, inc=1, device_id=(left,),
                            device_id_type=pl.DeviceIdType.MESH)
        pl.semaphore_signal(barrier, inc=1, device_id=(right,),
                            device_id_type=pl.DeviceIdType.MESH)
        pl.semaphore_wait(barrier, 2)

        x_bf[...] = x_ref[...].astype(BF16)
        w_bf[...] = w_ref[...].astype(BF16)

        def qpart(b, half, sub):
            xb = x_bf[pl.ds(b * MB + sub * SB, SB), :]
            wb = w_bf[:, 0:NH] if half == 0 else w_bf[:, NH:2 * NH]
            p = jnp.dot(xb, wb, preferred_element_type=F32)
            return jnp.round(p * SCALE_FP).astype(I16)

        def fpart(b, half, sub):
            xb = x_bf[pl.ds(b * MB + sub * SB, SB), :]
            wb = w_bf[:, 0:NH] if half == 0 else w_bf[:, NH:2 * NH]
            return jnp.dot(xb, wb, preferred_element_type=F32)

        b_cw = jnp.mod(d - 1, N_DEV)
        b_ccw = jnp.mod(d + 1, N_DEV)
        acc_cwA[0] = qpart(b_cw, 0, 0)
        acc_cwB[0] = qpart(b_cw, 0, 1)
        acc_ccwA[0] = qpart(b_ccw, 1, 0)
        acc_ccwB[0] = qpart(b_ccw, 1, 1)

        units = [
            ("cwA", acc_cwA, recv_cwA, ssem_cwA, rsem_cwA, cred_cwA),
            ("ccwA", acc_ccwA, recv_ccwA, ssem_ccwA, rsem_ccwA, cred_ccwA),
            ("cwB", acc_cwB, recv_cwB, ssem_cwB, rsem_cwB, cred_cwB),
            ("ccwB", acc_ccwB, recv_ccwB, ssem_ccwB, rsem_ccwB, cred_ccwB),
        ]
        send_to = {"cwA": right, "cwB": right, "ccwA": left, "ccwB": left}
        cred_to = {"cwA": left, "cwB": left, "ccwA": right, "ccwB": right}

        descs = {name: {} for name, *_ in units}
        for s in range(N_DEV - 1):
            a = s % 2
            if s >= 2:
                for name, _acc, _recv, _ss, _rs, cred in units:
                    pl.semaphore_wait(cred, 1)
            started = {}
            for name, acc, recv, ss, rs, _cred in units:
                rd = pltpu.make_async_remote_copy(
                    src_ref=acc.at[a],
                    dst_ref=recv.at[a],
                    send_sem=ss.at[a],
                    recv_sem=rs.at[a],
                    device_id=(send_to[name],),
                    device_id_type=pl.DeviceIdType.MESH,
                )
                rd.start()
                started[name] = rd
            nb_cw = jnp.mod(d - 2 - s, N_DEV)
            nb_ccw = jnp.mod(d + 2 + s, N_DEV)
            blk = {"cwA": (nb_cw, 0, 0), "cwB": (nb_cw, 0, 1),
                   "ccwA": (nb_ccw, 1, 0), "ccwB": (nb_ccw, 1, 1)}
            last = s == N_DEV - 2
            out_slc = {"cwA": (slice(0, SB), slice(0, NH)),
                       "cwB": (slice(SB, MB), slice(0, NH)),
                       "ccwA": (slice(0, SB), slice(NH, 2 * NH)),
                       "ccwB": (slice(SB, MB), slice(NH, 2 * NH))}
            for name, acc, recv, _ss, _rs, cred in units:
                b, half, sub = blk[name]
                if last:
                    p = fpart(b, half, sub)
                else:
                    qp = qpart(b, half, sub)
                if s >= 1:
                    descs[name][1 - a].wait_send()
                started[name].wait_recv()
                if last:
                    rs_slc, cs_slc = out_slc[name]
                    out_ref[rs_slc, cs_slc] = jnp.maximum(
                        recv[a].astype(F32) * (1.0 / SCALE_FP) + p, 0.0)
                else:
                    acc[1 - a] = recv[a] + qp
                if s + 2 <= N_DEV - 2:
                    pl.semaphore_signal(cred, inc=1,
                                        device_id=(cred_to[name],),
                                        device_id_type=pl.DeviceIdType.MESH)
                descs[name][a] = started[name]
        final_a = (N_DEV - 2) % 2
        for name, *_ in units:
            descs[name][final_a].wait_send()

        CH = 1024
        lmax = jnp.float32(0.0)
        for c in range(0, 2 * NH, CH):
            lmax = jnp.maximum(lmax, jnp.max(out_ref[:, c:c + CH]))
        amax_src[...] = jnp.full((1, 8, 128), lmax, F32)
        amax_buf[pl.ds(d, 1)] = amax_src[...]

        amax_sends = []
        for kk in range(1, N_DEV):
            j = jnp.mod(d + kk, N_DEV)
            rd = pltpu.make_async_remote_copy(
                src_ref=amax_src,
                dst_ref=amax_buf.at[pl.ds(d, 1)],
                send_sem=a_ssem.at[kk - 1],
                recv_sem=a_rsem.at[d],
                device_id=(j,),
                device_id_type=pl.DeviceIdType.MESH,
            )
            rd.start()
            amax_sends.append(rd)
        for kk in range(1, N_DEV):
            i = jnp.mod(d + kk, N_DEV)
            rr = pltpu.make_async_remote_copy(
                src_ref=amax_src,
                dst_ref=amax_buf.at[pl.ds(i, 1)],
                send_sem=a_ssem.at[0],
                recv_sem=a_rsem.at[i],
                device_id=(d,),
                device_id_type=pl.DeviceIdType.MESH,
            )
            rr.wait_recv()
        for rd in amax_sends:
            rd.wait_send()

        gmax = jnp.max(amax_buf[...])
        scale = gmax / 448.0
        for c in range(0, 2 * NH, CH):
            q = (out_ref[:, c:c + CH] / scale).astype(jnp.float8_e4m3fn)
            out_ref[:, c:c + CH] = q.astype(F32) * scale

    return pl.pallas_call(
        body,
        out_shape=jax.ShapeDtypeStruct((MB, n), F32),
        in_specs=[
            pl.BlockSpec(memory_space=pltpu.VMEM),
            pl.BlockSpec(memory_space=pltpu.VMEM),
        ],
        out_specs=pl.BlockSpec(memory_space=pltpu.VMEM),
        scratch_shapes=[
            pltpu.VMEM((4096, 256), BF16),
            pltpu.VMEM((256, 8192), BF16),
            pltpu.VMEM((2, SB, NH), I16),
            pltpu.VMEM((2, SB, NH), I16),
            pltpu.VMEM((2, SB, NH), I16),
            pltpu.VMEM((2, SB, NH), I16),
            pltpu.VMEM((2, SB, NH), I16),
            pltpu.VMEM((2, SB, NH), I16),
            pltpu.VMEM((2, SB, NH), I16),
            pltpu.VMEM((2, SB, NH), I16),
            pltpu.SemaphoreType.DMA((2,)),
            pltpu.SemaphoreType.DMA((2,)),
            pltpu.SemaphoreType.DMA((2,)),
            pltpu.SemaphoreType.DMA((2,)),
            pltpu.SemaphoreType.DMA((2,)),
            pltpu.SemaphoreType.DMA((2,)),
            pltpu.SemaphoreType.DMA((2,)),
            pltpu.SemaphoreType.DMA((2,)),
            pltpu.SemaphoreType.REGULAR,
            pltpu.SemaphoreType.REGULAR,
            pltpu.SemaphoreType.REGULAR,
            pltpu.SemaphoreType.REGULAR,
            pltpu.VMEM((1, 8, 128), F32),
            pltpu.VMEM((N_DEV, 8, 128), F32),
            pltpu.SemaphoreType.DMA((N_DEV - 1,)),
            pltpu.SemaphoreType.DMA((N_DEV,)),
        ],
        compiler_params=pltpu.CompilerParams(collective_id=0),
    )(x, w_mat)


# device time: 363278 ns/iter; 1.1609x vs baseline; 1.1391x over previous
import jax
import jax.numpy as jnp
from jax import lax
from jax.experimental import pallas as pl
from jax.experimental.pallas import tpu as pltpu

N_DEV = 16
MB = 256
SB = 128
NH = 4096
SCALE_FP = 2048.0
F32 = jnp.float32
BF16 = jnp.bfloat16
I16 = jnp.int16


def kernel(x, w_mat):
    m, k_sh = x.shape
    _, n = w_mat.shape

    def body(x_ref, w_ref, out_ref, x_bf, w_bf,
             acc_cwA, acc_cwB, acc_ccwA, acc_ccwB,
             recv_cwA, recv_cwB, recv_ccwA, recv_ccwB,
             ssem_cwA, rsem_cwA, ssem_cwB, rsem_cwB,
             ssem_ccwA, rsem_ccwA, ssem_ccwB, rsem_ccwB,
             cred_cwA, cred_cwB, cred_ccwA, cred_ccwB,
             amax_src, amax_buf, a_ssem, a_rsem):
        d = lax.axis_index("i")
        right = jnp.mod(d + 1, N_DEV)
        left = jnp.mod(d - 1, N_DEV)

        barrier = pltpu.get_barrier_semaphore()
        pl.semaphore_signal(barrier, inc=1, device_id=(left,),
                            device_id_type=pl.DeviceIdType.MESH)
        pl.semaphore_signal(barrier, inc=1, device_id=(right,),
                            device_id_type=pl.DeviceIdType.MESH)
        pl.semaphore_wait(barrier, 2)

        x_bf[...] = x_ref[...].astype(BF16)
        w_bf[...] = w_ref[...].astype(BF16)

        def qpart(b, half, sub):
            xb = x_bf[pl.ds(b * MB + sub * SB, SB), :]
            wb = w_bf[:, 0:NH] if half == 0 else w_bf[:, NH:2 * NH]
            p = jnp.dot(xb, wb, preferred_element_type=F32)
            return jnp.round(p * SCALE_FP).astype(I16)

        def fpart(b, half, sub):
            xb = x_bf[pl.ds(b * MB + sub * SB, SB), :]
            wb = w_bf[:, 0:NH] if half == 0 else w_bf[:, NH:2 * NH]
            return jnp.dot(xb, wb, preferred_element_type=F32)

        b_cw = jnp.mod(d - 1, N_DEV)
        b_ccw = jnp.mod(d + 1, N_DEV)
        acc_cwA[0] = qpart(b_cw, 0, 0)
        acc_cwB[0] = qpart(b_cw, 0, 1)
        acc_ccwA[0] = qpart(b_ccw, 1, 0)
        acc_ccwB[0] = qpart(b_ccw, 1, 1)

        units = [
            ("cwA", acc_cwA, recv_cwA, ssem_cwA, rsem_cwA, cred_cwA),
            ("ccwA", acc_ccwA, recv_ccwA, ssem_ccwA, rsem_ccwA, cred_ccwA),
            ("cwB", acc_cwB, recv_cwB, ssem_cwB, rsem_cwB, cred_cwB),
            ("ccwB", acc_ccwB, recv_ccwB, ssem_ccwB, rsem_ccwB, cred_ccwB),
        ]
        send_to = {"cwA": right, "cwB": right, "ccwA": left, "ccwB": left}
        cred_to = {"cwA": left, "cwB": left, "ccwA": right, "ccwB": right}

        out_slc = {"cwA": (slice(0, SB), slice(0, NH)),
                   "cwB": (slice(SB, MB), slice(0, NH)),
                   "ccwA": (slice(0, SB), slice(NH, 2 * NH)),
                   "ccwB": (slice(SB, MB), slice(NH, 2 * NH))}

        def start_unit(u, s):
            name, acc, recv, ss, rs, _cred = u
            a = s % 2
            rd = pltpu.make_async_remote_copy(
                src_ref=acc.at[a],
                dst_ref=recv.at[a],
                send_sem=ss.at[a],
                recv_sem=rs.at[a],
                device_id=(send_to[name],),
                device_id_type=pl.DeviceIdType.MESH,
            )
            rd.start()
            return rd

        descs = {name: {} for name, *_ in units}
        cur = {}
        for u in units:
            cur[u[0]] = start_unit(u, 0)
        for s in range(N_DEV - 1):
            a = s % 2
            nb_cw = jnp.mod(d - 2 - s, N_DEV)
            nb_ccw = jnp.mod(d + 2 + s, N_DEV)
            blk = {"cwA": (nb_cw, 0, 0), "cwB": (nb_cw, 0, 1),
                   "ccwA": (nb_ccw, 1, 0), "ccwB": (nb_ccw, 1, 1)}
            last = s == N_DEV - 2
            for u in units:
                name, acc, recv, _ss, _rs, cred = u
                b, half, sub = blk[name]
                if last:
                    p = fpart(b, half, sub)
                else:
                    qp = qpart(b, half, sub)
                if s >= 1:
                    descs[name][1 - a].wait_send()
                cur[name].wait_recv()
                if last:
                    rs_slc, cs_slc = out_slc[name]
                    out_ref[rs_slc, cs_slc] = jnp.maximum(
                        recv[a].astype(F32) * (1.0 / SCALE_FP) + p, 0.0)
                else:
                    acc[1 - a] = recv[a] + qp
                if s + 2 <= N_DEV - 2:
                    pl.semaphore_signal(cred, inc=1,
                                        device_id=(cred_to[name],),
                                        device_id_type=pl.DeviceIdType.MESH)
                descs[name][a] = cur[name]
                if not last:
                    if s + 1 >= 2:
                        pl.semaphore_wait(cred, 1)
                    cur[name] = start_unit(u, s + 1)
        final_a = (N_DEV - 2) % 2
        for name, *_ in units:
            descs[name][final_a].wait_send()

        CH = 1024
        lmax = jnp.float32(0.0)
        for c in range(0, 2 * NH, CH):
            lmax = jnp.maximum(lmax, jnp.max(out_ref[:, c:c + CH]))
        amax_src[...] = jnp.full((1, 8, 128), lmax, F32)
        amax_buf[pl.ds(d, 1)] = amax_src[...]

        amax_sends = []
        for kk in range(1, N_DEV):
            j = jnp.mod(d + kk, N_DEV)
            rd = pltpu.make_async_remote_copy(
                src_ref=amax_src,
                dst_ref=amax_buf.at[pl.ds(d, 1)],
                send_sem=a_ssem.at[kk - 1],
                recv_sem=a_rsem.at[d],
                device_id=(j,),
                device_id_type=pl.DeviceIdType.MESH,
            )
            rd.start()
            amax_sends.append(rd)
        for kk in range(1, N_DEV):
            i = jnp.mod(d + kk, N_DEV)
            rr = pltpu.make_async_remote_copy(
                src_ref=amax_src,
                dst_ref=amax_buf.at[pl.ds(i, 1)],
                send_sem=a_ssem.at[0],
                recv_sem=a_rsem.at[i],
                device_id=(d,),
                device_id_type=pl.DeviceIdType.MESH,
            )
            rr.wait_recv()
        for rd in amax_sends:
            rd.wait_send()

        gmax = jnp.max(amax_buf[...])
        scale = gmax / 448.0
        for c in range(0, 2 * NH, CH):
            q = (out_ref[:, c:c + CH] / scale).astype(jnp.float8_e4m3fn)
            out_ref[:, c:c + CH] = q.astype(F32) * scale

    return pl.pallas_call(
        body,
        out_shape=jax.ShapeDtypeStruct((MB, n), F32),
        in_specs=[
            pl.BlockSpec(memory_space=pltpu.VMEM),
            pl.BlockSpec(memory_space=pltpu.VMEM),
        ],
        out_specs=pl.BlockSpec(memory_space=pltpu.VMEM),
        scratch_shapes=[
            pltpu.VMEM((4096, 256), BF16),
            pltpu.VMEM((256, 8192), BF16),
            pltpu.VMEM((2, SB, NH), I16),
            pltpu.VMEM((2, SB, NH), I16),
            pltpu.VMEM((2, SB, NH), I16),
            pltpu.VMEM((2, SB, NH), I16),
            pltpu.VMEM((2, SB, NH), I16),
            pltpu.VMEM((2, SB, NH), I16),
            pltpu.VMEM((2, SB, NH), I16),
            pltpu.VMEM((2, SB, NH), I16),
            pltpu.SemaphoreType.DMA((2,)),
            pltpu.SemaphoreType.DMA((2,)),
            pltpu.SemaphoreType.DMA((2,)),
            pltpu.SemaphoreType.DMA((2,)),
            pltpu.SemaphoreType.DMA((2,)),
            pltpu.SemaphoreType.DMA((2,)),
            pltpu.SemaphoreType.DMA((2,)),
            pltpu.SemaphoreType.DMA((2,)),
            pltpu.SemaphoreType.REGULAR,
            pltpu.SemaphoreType.REGULAR,
            pltpu.SemaphoreType.REGULAR,
            pltpu.SemaphoreType.REGULAR,
            pltpu.VMEM((1, 8, 128), F32),
            pltpu.VMEM((N_DEV, 8, 128), F32),
            pltpu.SemaphoreType.DMA((N_DEV - 1,)),
            pltpu.SemaphoreType.DMA((N_DEV,)),
        ],
        compiler_params=pltpu.CompilerParams(collective_id=0),
    )(x, w_mat)


# device time: 363063 ns/iter; 1.1616x vs baseline; 1.0006x over previous
import jax
import jax.numpy as jnp
from jax import lax
from jax.experimental import pallas as pl
from jax.experimental.pallas import tpu as pltpu

N_DEV = 16
MB = 256
SB = 128
NH = 4096
SCALE_FP = 2048.0
F32 = jnp.float32
BF16 = jnp.bfloat16
I16 = jnp.int16


def kernel(x, w_mat):
    m, k_sh = x.shape
    _, n = w_mat.shape

    def body(x_ref, w_ref, out_ref, x_bf, w_bf,
             acc_cwA, acc_cwB, acc_ccwA, acc_ccwB,
             recv_cwA, recv_cwB, recv_ccwA, recv_ccwB,
             ssem_cwA, rsem_cwA, ssem_cwB, rsem_cwB,
             ssem_ccwA, rsem_ccwA, ssem_ccwB, rsem_ccwB,
             cred_cwA, cred_cwB, cred_ccwA, cred_ccwB,
             amax_src, amax_buf, a_ssem, a_rsem):
        d = lax.axis_index("i")
        right = jnp.mod(d + 1, N_DEV)
        left = jnp.mod(d - 1, N_DEV)

        barrier = pltpu.get_barrier_semaphore()
        pl.semaphore_signal(barrier, inc=1, device_id=(left,),
                            device_id_type=pl.DeviceIdType.MESH)
        pl.semaphore_signal(barrier, inc=1, device_id=(right,),
                            device_id_type=pl.DeviceIdType.MESH)
        pl.semaphore_wait(barrier, 2)

        x_bf[...] = x_ref[...].astype(BF16)
        w_bf[...] = w_ref[...].astype(BF16)

        def qpart(b, half, sub):
            xb = x_bf[pl.ds(b * MB + sub * SB, SB), :]
            wb = w_bf[:, 0:NH] if half == 0 else w_bf[:, NH:2 * NH]
            p = jnp.dot(xb, wb, preferred_element_type=F32)
            return jnp.round(p * SCALE_FP).astype(I16)

        def fpart(b, half, sub):
            xb = x_bf[pl.ds(b * MB + sub * SB, SB), :]
            wb = w_bf[:, 0:NH] if half == 0 else w_bf[:, NH:2 * NH]
            return jnp.dot(xb, wb, preferred_element_type=F32)

        b_cw = jnp.mod(d - 1, N_DEV)
        b_ccw = jnp.mod(d + 1, N_DEV)
        acc_cwA[0] = qpart(b_cw, 0, 0)
        acc_cwB[0] = qpart(b_cw, 0, 1)
        acc_ccwA[0] = qpart(b_ccw, 1, 0)
        acc_ccwB[0] = qpart(b_ccw, 1, 1)

        units = [
            ("cwA", acc_cwA, recv_cwA, ssem_cwA, rsem_cwA, cred_cwA),
            ("ccwA", acc_ccwA, recv_ccwA, ssem_ccwA, rsem_ccwA, cred_ccwA),
            ("cwB", acc_cwB, recv_cwB, ssem_cwB, rsem_cwB, cred_cwB),
            ("ccwB", acc_ccwB, recv_ccwB, ssem_ccwB, rsem_ccwB, cred_ccwB),
        ]
        send_to = {"cwA": right, "cwB": right, "ccwA": left, "ccwB": left}
        cred_to = {"cwA": left, "cwB": left, "ccwA": right, "ccwB": right}

        out_slc = {"cwA": (slice(0, SB), slice(0, NH)),
                   "cwB": (slice(SB, MB), slice(0, NH)),
                   "ccwA": (slice(0, SB), slice(NH, 2 * NH)),
                   "ccwB": (slice(SB, MB), slice(NH, 2 * NH))}

        def start_unit(u, s):
            name, acc, recv, ss, rs, _cred = u
            a = s % 2
            rd = pltpu.make_async_remote_copy(
                src_ref=acc.at[a],
                dst_ref=recv.at[a],
                send_sem=ss.at[a],
                recv_sem=rs.at[a],
                device_id=(send_to[name],),
                device_id_type=pl.DeviceIdType.MESH,
            )
            rd.start()
            return rd

        descs = {name: {} for name, *_ in units}
        cur = {}
        for u in units:
            cur[u[0]] = start_unit(u, 0)
        lmax = jnp.float32(0.0)
        for s in range(N_DEV - 1):
            a = s % 2
            nb_cw = jnp.mod(d - 2 - s, N_DEV)
            nb_ccw = jnp.mod(d + 2 + s, N_DEV)
            blk = {"cwA": (nb_cw, 0, 0), "cwB": (nb_cw, 0, 1),
                   "ccwA": (nb_ccw, 1, 0), "ccwB": (nb_ccw, 1, 1)}
            last = s == N_DEV - 2
            for u in units:
                name, acc, recv, _ss, _rs, cred = u
                b, half, sub = blk[name]
                if last:
                    p = fpart(b, half, sub)
                else:
                    qp = qpart(b, half, sub)
                if s >= 1:
                    descs[name][1 - a].wait_send()
                cur[name].wait_recv()
                if last:
                    rs_slc, cs_slc = out_slc[name]
                    y = jnp.maximum(
                        recv[a].astype(F32) * (1.0 / SCALE_FP) + p, 0.0)
                    out_ref[rs_slc, cs_slc] = y
                    lmax = jnp.maximum(lmax, jnp.max(y))
                else:
                    acc[1 - a] = recv[a] + qp
                if s + 2 <= N_DEV - 2:
                    pl.semaphore_signal(cred, inc=1,
                                        device_id=(cred_to[name],),
                                        device_id_type=pl.DeviceIdType.MESH)
                descs[name][a] = cur[name]
                if not last:
                    if s + 1 >= 2:
                        pl.semaphore_wait(cred, 1)
                    cur[name] = start_unit(u, s + 1)
        final_a = (N_DEV - 2) % 2
        for name, *_ in units:
            descs[name][final_a].wait_send()

        CH = 2048
        amax_src[...] = jnp.full((1, 8, 128), lmax, F32)
        amax_buf[pl.ds(d, 1)] = amax_src[...]

        amax_sends = []
        for kk in range(1, N_DEV):
            j = jnp.mod(d + kk, N_DEV)
            rd = pltpu.make_async_remote_copy(
                src_ref=amax_src,
                dst_ref=amax_buf.at[pl.ds(d, 1)],
                send_sem=a_ssem.at[kk - 1],
                recv_sem=a_rsem.at[d],
                device_id=(j,),
                device_id_type=pl.DeviceIdType.MESH,
            )
            rd.start()
            amax_sends.append(rd)
        for kk in range(1, N_DEV):
            i = jnp.mod(d + kk, N_DEV)
            rr = pltpu.make_async_remote_copy(
                src_ref=amax_src,
                dst_ref=amax_buf.at[pl.ds(i, 1)],
                send_sem=a_ssem.at[0],
                recv_sem=a_rsem.at[i],
                device_id=(d,),
                device_id_type=pl.DeviceIdType.MESH,
            )
            rr.wait_recv()
        for rd in amax_sends:
            rd.wait_send()

        gmax = jnp.max(amax_buf[...])
        scale = gmax / 448.0
        inv_scale = 1.0 / scale
        for c in range(0, 2 * NH, CH):
            q = (out_ref[:, c:c + CH] * inv_scale).astype(jnp.float8_e4m3fn)
            out_ref[:, c:c + CH] = q.astype(F32) * scale

    return pl.pallas_call(
        body,
        out_shape=jax.ShapeDtypeStruct((MB, n), F32),
        in_specs=[
            pl.BlockSpec(memory_space=pltpu.VMEM),
            pl.BlockSpec(memory_space=pltpu.VMEM),
        ],
        out_specs=pl.BlockSpec(memory_space=pltpu.VMEM),
        scratch_shapes=[
            pltpu.VMEM((4096, 256), BF16),
            pltpu.VMEM((256, 8192), BF16),
            pltpu.VMEM((2, SB, NH), I16),
            pltpu.VMEM((2, SB, NH), I16),
            pltpu.VMEM((2, SB, NH), I16),
            pltpu.VMEM((2, SB, NH), I16),
            pltpu.VMEM((2, SB, NH), I16),
            pltpu.VMEM((2, SB, NH), I16),
            pltpu.VMEM((2, SB, NH), I16),
            pltpu.VMEM((2, SB, NH), I16),
            pltpu.SemaphoreType.DMA((2,)),
            pltpu.SemaphoreType.DMA((2,)),
            pltpu.SemaphoreType.DMA((2,)),
            pltpu.SemaphoreType.DMA((2,)),
            pltpu.SemaphoreType.DMA((2,)),
            pltpu.SemaphoreType.DMA((2,)),
            pltpu.SemaphoreType.DMA((2,)),
            pltpu.SemaphoreType.DMA((2,)),
            pltpu.SemaphoreType.REGULAR,
            pltpu.SemaphoreType.REGULAR,
            pltpu.SemaphoreType.REGULAR,
            pltpu.SemaphoreType.REGULAR,
            pltpu.VMEM((1, 8, 128), F32),
            pltpu.VMEM((N_DEV, 8, 128), F32),
            pltpu.SemaphoreType.DMA((N_DEV - 1,)),
            pltpu.SemaphoreType.DMA((N_DEV,)),
        ],
        compiler_params=pltpu.CompilerParams(collective_id=0),
    )(x, w_mat)


# device time: 363023 ns/iter; 1.1617x vs baseline; 1.0001x over previous
import jax
import jax.numpy as jnp
from jax import lax
from jax.experimental import pallas as pl
from jax.experimental.pallas import tpu as pltpu

N_DEV = 16
MB = 256
SB = 128
NH = 4096
SCALE_FP = 2048.0
F32 = jnp.float32
BF16 = jnp.bfloat16
I16 = jnp.int16


def kernel(x, w_mat):
    m, k_sh = x.shape
    _, n = w_mat.shape

    def body(x_ref, w_ref, out_ref, x_bf, w_bf,
             acc_cwA, acc_cwB, acc_ccwA, acc_ccwB,
             recv_cwA, recv_cwB, recv_ccwA, recv_ccwB,
             ssem_cwA, rsem_cwA, ssem_cwB, rsem_cwB,
             ssem_ccwA, rsem_ccwA, ssem_ccwB, rsem_ccwB,
             cred_cwA, cred_cwB, cred_ccwA, cred_ccwB,
             amax_src, amax_buf, a_ssem, a_rsem):
        d = lax.axis_index("i")
        right = jnp.mod(d + 1, N_DEV)
        left = jnp.mod(d - 1, N_DEV)

        barrier = pltpu.get_barrier_semaphore()
        pl.semaphore_signal(barrier, inc=1, device_id=(left,),
                            device_id_type=pl.DeviceIdType.MESH)
        pl.semaphore_signal(barrier, inc=1, device_id=(right,),
                            device_id_type=pl.DeviceIdType.MESH)
        pl.semaphore_wait(barrier, 2)

        w_bf[...] = w_ref[...].astype(BF16)

        def qpart(b, half, sub):
            xb = x_bf[pl.ds(b * MB + sub * SB, SB), :]
            wb = w_bf[:, 0:NH] if half == 0 else w_bf[:, NH:2 * NH]
            p = jnp.dot(xb, wb, preferred_element_type=F32)
            return jnp.round(p * SCALE_FP).astype(I16)

        def fpart(b, half, sub):
            xb = x_bf[pl.ds(b * MB + sub * SB, SB), :]
            wb = w_bf[:, 0:NH] if half == 0 else w_bf[:, NH:2 * NH]
            return jnp.dot(xb, wb, preferred_element_type=F32)

        b_cw = jnp.mod(d - 1, N_DEV)
        b_ccw = jnp.mod(d + 1, N_DEV)

        units = [
            ("cwA", acc_cwA, recv_cwA, ssem_cwA, rsem_cwA, cred_cwA),
            ("ccwA", acc_ccwA, recv_ccwA, ssem_ccwA, rsem_ccwA, cred_ccwA),
            ("cwB", acc_cwB, recv_cwB, ssem_cwB, rsem_cwB, cred_cwB),
            ("ccwB", acc_ccwB, recv_ccwB, ssem_ccwB, rsem_ccwB, cred_ccwB),
        ]
        send_to = {"cwA": right, "cwB": right, "ccwA": left, "ccwB": left}
        cred_to = {"cwA": left, "cwB": left, "ccwA": right, "ccwB": right}

        out_slc = {"cwA": (slice(0, SB), slice(0, NH)),
                   "cwB": (slice(SB, MB), slice(0, NH)),
                   "ccwA": (slice(0, SB), slice(NH, 2 * NH)),
                   "ccwB": (slice(SB, MB), slice(NH, 2 * NH))}

        def start_unit(u, s):
            name, acc, recv, ss, rs, _cred = u
            a = s % 2
            rd = pltpu.make_async_remote_copy(
                src_ref=acc.at[a],
                dst_ref=recv.at[a],
                send_sem=ss.at[a],
                recv_sem=rs.at[a],
                device_id=(send_to[name],),
                device_id_type=pl.DeviceIdType.MESH,
            )
            rd.start()
            return rd

        descs = {name: {} for name, *_ in units}
        cur = {}
        x_bf[pl.ds(b_cw * MB, MB), :] = (
            x_ref[pl.ds(b_cw * MB, MB), :].astype(BF16))
        acc_cwA[0] = qpart(b_cw, 0, 0)
        cur["cwA"] = start_unit(units[0], 0)
        acc_cwB[0] = qpart(b_cw, 0, 1)
        cur["cwB"] = start_unit(units[2], 0)
        x_bf[pl.ds(b_ccw * MB, MB), :] = (
            x_ref[pl.ds(b_ccw * MB, MB), :].astype(BF16))
        acc_ccwA[0] = qpart(b_ccw, 1, 0)
        cur["ccwA"] = start_unit(units[1], 0)
        acc_ccwB[0] = qpart(b_ccw, 1, 1)
        cur["ccwB"] = start_unit(units[3], 0)
        x_bf[...] = x_ref[...].astype(BF16)
        lmax = jnp.float32(0.0)
        for s in range(N_DEV - 1):
            a = s % 2
            nb_cw = jnp.mod(d - 2 - s, N_DEV)
            nb_ccw = jnp.mod(d + 2 + s, N_DEV)
            blk = {"cwA": (nb_cw, 0, 0), "cwB": (nb_cw, 0, 1),
                   "ccwA": (nb_ccw, 1, 0), "ccwB": (nb_ccw, 1, 1)}
            last = s == N_DEV - 2
            for u in units:
                name, acc, recv, _ss, _rs, cred = u
                b, half, sub = blk[name]
                if last:
                    p = fpart(b, half, sub)
                else:
                    qp = qpart(b, half, sub)
                if s >= 1:
                    descs[name][1 - a].wait_send()
                cur[name].wait_recv()
                if last:
                    rs_slc, cs_slc = out_slc[name]
                    y = jnp.maximum(
                        recv[a].astype(F32) * (1.0 / SCALE_FP) + p, 0.0)
                    out_ref[rs_slc, cs_slc] = y
                    lmax = jnp.maximum(lmax, jnp.max(y))
                else:
                    acc[1 - a] = recv[a] + qp
                if s + 2 <= N_DEV - 2:
                    pl.semaphore_signal(cred, inc=1,
                                        device_id=(cred_to[name],),
                                        device_id_type=pl.DeviceIdType.MESH)
                descs[name][a] = cur[name]
                if not last:
                    if s + 1 >= 2:
                        pl.semaphore_wait(cred, 1)
                    cur[name] = start_unit(u, s + 1)
        final_a = (N_DEV - 2) % 2
        for name, *_ in units:
            descs[name][final_a].wait_send()

        CH = 2048
        amax_src[...] = jnp.full((1, 8, 128), lmax, F32)
        amax_buf[pl.ds(d, 1)] = amax_src[...]

        amax_sends = []
        for kk in range(1, N_DEV):
            j = jnp.mod(d + kk, N_DEV)
            rd = pltpu.make_async_remote_copy(
                src_ref=amax_src,
                dst_ref=amax_buf.at[pl.ds(d, 1)],
                send_sem=a_ssem.at[kk - 1],
                recv_sem=a_rsem.at[d],
                device_id=(j,),
                device_id_type=pl.DeviceIdType.MESH,
            )
            rd.start()
            amax_sends.append(rd)
        for kk in range(1, N_DEV):
            i = jnp.mod(d + kk, N_DEV)
            rr = pltpu.make_async_remote_copy(
                src_ref=amax_src,
                dst_ref=amax_buf.at[pl.ds(i, 1)],
                send_sem=a_ssem.at[0],
                recv_sem=a_rsem.at[i],
                device_id=(d,),
                device_id_type=pl.DeviceIdType.MESH,
            )
            rr.wait_recv()
        for rd in amax_sends:
            rd.wait_send()

        gmax = jnp.max(amax_buf[...])
        scale = gmax / 448.0
        inv_scale = 1.0 / scale
        for c in range(0, 2 * NH, CH):
            q = (out_ref[:, c:c + CH] * inv_scale).astype(jnp.float8_e4m3fn)
            out_ref[:, c:c + CH] = q.astype(F32) * scale

    return pl.pallas_call(
        body,
        out_shape=jax.ShapeDtypeStruct((MB, n), F32),
        in_specs=[
            pl.BlockSpec(memory_space=pltpu.VMEM),
            pl.BlockSpec(memory_space=pltpu.VMEM),
        ],
        out_specs=pl.BlockSpec(memory_space=pltpu.VMEM),
        scratch_shapes=[
            pltpu.VMEM((4096, 256), BF16),
            pltpu.VMEM((256, 8192), BF16),
            pltpu.VMEM((2, SB, NH), I16),
            pltpu.VMEM((2, SB, NH), I16),
            pltpu.VMEM((2, SB, NH), I16),
            pltpu.VMEM((2, SB, NH), I16),
            pltpu.VMEM((2, SB, NH), I16),
            pltpu.VMEM((2, SB, NH), I16),
            pltpu.VMEM((2, SB, NH), I16),
            pltpu.VMEM((2, SB, NH), I16),
            pltpu.SemaphoreType.DMA((2,)),
            pltpu.SemaphoreType.DMA((2,)),
            pltpu.SemaphoreType.DMA((2,)),
            pltpu.SemaphoreType.DMA((2,)),
            pltpu.SemaphoreType.DMA((2,)),
            pltpu.SemaphoreType.DMA((2,)),
            pltpu.SemaphoreType.DMA((2,)),
            pltpu.SemaphoreType.DMA((2,)),
            pltpu.SemaphoreType.REGULAR,
            pltpu.SemaphoreType.REGULAR,
            pltpu.SemaphoreType.REGULAR,
            pltpu.SemaphoreType.REGULAR,
            pltpu.VMEM((1, 8, 128), F32),
            pltpu.VMEM((N_DEV, 8, 128), F32),
            pltpu.SemaphoreType.DMA((N_DEV - 1,)),
            pltpu.SemaphoreType.DMA((N_DEV,)),
        ],
        compiler_params=pltpu.CompilerParams(collective_id=0),
    )(x, w_mat)
